# baseline (device time: 98998 ns/iter reference)
import jax
import jax.numpy as jnp
from jax import lax
from jax.experimental import pallas as pl
from jax.experimental.pallas import tpu as pltpu

CHUNK = 256


def kernel(x, W):
    T, D = x.shape
    _, V = W.shape
    n_chunks = V // CHUNK
    nq = n_chunks // 2

    def body_a(
        x_ref, w_hbm,
        eloc_ref, rx_ref, ry_ref, s_ref,
        w_buf, w_sems, xsend_sems, xrecv_sems, fsend_sems, yrecv_sems,
    ):
        my_x = lax.axis_index("x")
        my_y = lax.axis_index("y")
        partner = (1 - my_x, my_y)
        neighbor = (my_x, 1 - my_y)

        barrier = pltpu.get_barrier_semaphore()
        for nbr in (partner, neighbor):
            pl.semaphore_signal(
                barrier, inc=1, device_id=nbr,
                device_id_type=pl.DeviceIdType.MESH,
            )
        pl.semaphore_wait(barrier, 2)

        xb = x_ref[...].astype(jnp.bfloat16)

        def fetch_w(c, slot):
            return pltpu.make_async_copy(
                w_hbm.at[:, pl.ds(c * CHUNK, CHUNK)],
                w_buf.at[slot],
                w_sems.at[slot],
            )

        def x_rdma(i):
            return pltpu.make_async_remote_copy(
                src_ref=eloc_ref.at[my_y * nq + i],
                dst_ref=rx_ref.at[i],
                send_sem=xsend_sems.at[i],
                recv_sem=xrecv_sems.at[i],
                device_id=partner,
                device_id_type=pl.DeviceIdType.MESH,
            )

        def y_rdma(i):
            return pltpu.make_async_remote_copy(
                src_ref=rx_ref.at[i],
                dst_ref=ry_ref.at[i],
                send_sem=fsend_sems.at[i],
                recv_sem=yrecv_sems.at[i],
                device_id=neighbor,
                device_id_type=pl.DeviceIdType.MESH,
            )

        def chunk_of(k):
            return jnp.where(
                k < nq, my_y * nq + k, (1 - my_y) * nq + (k - nq)
            )

        def compute_chunk(c):
            fetch_w(c, lax.rem(c, 2)).wait()
            wb = w_buf[lax.rem(c, 2)].astype(jnp.bfloat16)
            lc = jnp.dot(xb, wb, preferred_element_type=jnp.float32)
            e = jnp.exp(lc)
            eloc_ref[c] = e.astype(jnp.bfloat16)
            return e

        fetch_w(chunk_of(0), lax.rem(chunk_of(0), 2)).start()

        def send_step(k, s_own):
            @pl.when(k + 1 < n_chunks)
            def _():
                c2 = chunk_of(k + 1)
                fetch_w(c2, lax.rem(c2, 2)).start()

            e = compute_chunk(chunk_of(k))
            x_rdma(k).start()
            return s_own + jnp.sum(e, axis=1, keepdims=True)

        s_own = lax.fori_loop(
            0, nq, send_step, jnp.zeros((T, 1), jnp.float32)
        )

        def keep_step(k, carry):
            s_own, s_p = carry
            kk = k + nq

            @pl.when(kk + 1 < n_chunks)
            def _():
                c2 = chunk_of(kk + 1)
                fetch_w(c2, lax.rem(c2, 2)).start()

            e = compute_chunk(chunk_of(kk))
            s_own = s_own + jnp.sum(e, axis=1, keepdims=True)

            x_rdma(k).wait_recv()
            y_rdma(k).start()
            ep = rx_ref[k].astype(jnp.float32)
            return s_own, s_p + jnp.sum(ep, axis=1, keepdims=True)

        s_own, s_p = lax.fori_loop(
            0, nq, keep_step, (s_own, jnp.zeros((T, 1), jnp.float32))
        )

        def yrecv_step(i, acc):
            y_rdma(i).wait_recv()
            e = ry_ref[i].astype(jnp.float32)
            return acc + jnp.sum(e, axis=1, keepdims=True)

        s = s_own + lax.fori_loop(0, nq, yrecv_step, s_p)
        s_ref[...] = jnp.broadcast_to(s, (T, 128))

        def drain(i, carry):
            x_rdma(i).wait_send()
            y_rdma(i).wait_send()
            return carry

        lax.fori_loop(0, nq, drain, 0)

    eloc, rx, ry, svec = pl.pallas_call(
        body_a,
        out_shape=(
            jax.ShapeDtypeStruct((n_chunks, T, CHUNK), jnp.bfloat16),
            jax.ShapeDtypeStruct((nq, T, CHUNK), jnp.bfloat16),
            jax.ShapeDtypeStruct((nq, T, CHUNK), jnp.bfloat16),
            jax.ShapeDtypeStruct((T, 128), jnp.float32),
        ),
        in_specs=[
            pl.BlockSpec(memory_space=pltpu.VMEM),
            pl.BlockSpec(memory_space=pl.ANY),
        ],
        out_specs=(
            pl.BlockSpec(memory_space=pltpu.VMEM),
            pl.BlockSpec(memory_space=pltpu.VMEM),
            pl.BlockSpec(memory_space=pltpu.VMEM),
            pl.BlockSpec(memory_space=pltpu.VMEM),
        ),
        scratch_shapes=[
            pltpu.VMEM((2, D, CHUNK), jnp.float32),
            pltpu.SemaphoreType.DMA((2,)),
            pltpu.SemaphoreType.DMA((nq,)),
            pltpu.SemaphoreType.DMA((nq,)),
            pltpu.SemaphoreType.DMA((nq,)),
            pltpu.SemaphoreType.DMA((nq,)),
        ],
        compiler_params=pltpu.CompilerParams(
            collective_id=0,
            vmem_limit_bytes=62 * 1024 * 1024,
        ),
    )(x, W)

    def body_b(eloc_ref, rx_ref, ry_ref, s_ref, out_ref):
        my_x = lax.axis_index("x")
        my_y = lax.axis_index("y")
        my_base = my_x * V
        partner_base = (1 - my_x) * V

        r = 1.0 / s_ref[:, 0:1]

        def norm_local(c, carry):
            sl = pl.ds(my_base + c * CHUNK, CHUNK)
            out_ref[:, sl] = eloc_ref[c].astype(jnp.float32) * r
            return carry

        lax.fori_loop(0, n_chunks, norm_local, 0)

        def norm_partner(i, carry):
            xc = pl.ds(partner_base + (my_y * nq + i) * CHUNK, CHUNK)
            out_ref[:, xc] = rx_ref[i].astype(jnp.float32) * r
            yc = pl.ds(partner_base + ((1 - my_y) * nq + i) * CHUNK, CHUNK)
            out_ref[:, yc] = ry_ref[i].astype(jnp.float32) * r
            return carry

        lax.fori_loop(0, nq, norm_partner, 0)

    return pl.pallas_call(
        body_b,
        out_shape=jax.ShapeDtypeStruct((T, 2 * V), jnp.float32),
        in_specs=[
            pl.BlockSpec(memory_space=pltpu.VMEM),
            pl.BlockSpec(memory_space=pltpu.VMEM),
            pl.BlockSpec(memory_space=pltpu.VMEM),
            pl.BlockSpec(memory_space=pltpu.VMEM),
        ],
        out_specs=pl.BlockSpec(memory_space=pltpu.VMEM),
        compiler_params=pltpu.CompilerParams(
            vmem_limit_bytes=62 * 1024 * 1024,
        ),
    )(eloc, rx, ry, svec)


# device time: 91581 ns/iter; 1.0810x vs baseline; 1.0810x over previous
import jax
import jax.numpy as jnp
from jax import lax
from jax.experimental import pallas as pl
from jax.experimental.pallas import tpu as pltpu

CHUNK = 512


def kernel(x, W):
    T, D = x.shape
    _, V = W.shape
    n_chunks = V // CHUNK
    nq = n_chunks // 2

    def body_a(
        x_ref, w_hbm,
        eloc_ref, rx_ref, ry_ref, s_ref,
        w_buf, w_sems, xsend_sems, xrecv_sems, fsend_sems, yrecv_sems,
    ):
        my_x = lax.axis_index("x")
        my_y = lax.axis_index("y")
        partner = (1 - my_x, my_y)
        neighbor = (my_x, 1 - my_y)

        barrier = pltpu.get_barrier_semaphore()
        for nbr in (partner, neighbor):
            pl.semaphore_signal(
                barrier, inc=1, device_id=nbr,
                device_id_type=pl.DeviceIdType.MESH,
            )
        pl.semaphore_wait(barrier, 2)

        xb = x_ref[...].astype(jnp.bfloat16)

        def fetch_w(c, slot):
            return pltpu.make_async_copy(
                w_hbm.at[:, pl.ds(c * CHUNK, CHUNK)],
                w_buf.at[slot],
                w_sems.at[slot],
            )

        def x_rdma(i):
            return pltpu.make_async_remote_copy(
                src_ref=eloc_ref.at[my_y * nq + i],
                dst_ref=rx_ref.at[i],
                send_sem=xsend_sems.at[i],
                recv_sem=xrecv_sems.at[i],
                device_id=partner,
                device_id_type=pl.DeviceIdType.MESH,
            )

        def y_rdma(i):
            return pltpu.make_async_remote_copy(
                src_ref=rx_ref.at[i],
                dst_ref=ry_ref.at[i],
                send_sem=fsend_sems.at[i],
                recv_sem=yrecv_sems.at[i],
                device_id=neighbor,
                device_id_type=pl.DeviceIdType.MESH,
            )

        def chunk_of(k):
            return jnp.where(
                k < nq, my_y * nq + k, (1 - my_y) * nq + (k - nq)
            )

        def compute_chunk(c):
            fetch_w(c, lax.rem(c, 2)).wait()
            wb = w_buf[lax.rem(c, 2)].astype(jnp.bfloat16)
            lc = jnp.dot(xb, wb, preferred_element_type=jnp.float32)
            e = jnp.exp(lc)
            eloc_ref[c] = e.astype(jnp.bfloat16)
            return e

        fetch_w(chunk_of(0), lax.rem(chunk_of(0), 2)).start()

        def send_step(k, s_own):
            @pl.when(k + 1 < n_chunks)
            def _():
                c2 = chunk_of(k + 1)
                fetch_w(c2, lax.rem(c2, 2)).start()

            e = compute_chunk(chunk_of(k))
            x_rdma(k).start()
            return s_own + jnp.sum(e, axis=1, keepdims=True)

        s_own = lax.fori_loop(
            0, nq, send_step, jnp.zeros((T, 1), jnp.float32)
        )

        def keep_step(k, carry):
            s_own, s_p = carry
            kk = k + nq

            @pl.when(kk + 1 < n_chunks)
            def _():
                c2 = chunk_of(kk + 1)
                fetch_w(c2, lax.rem(c2, 2)).start()

            e = compute_chunk(chunk_of(kk))
            s_own = s_own + jnp.sum(e, axis=1, keepdims=True)

            x_rdma(k).wait_recv()
            y_rdma(k).start()
            ep = rx_ref[k].astype(jnp.float32)
            return s_own, s_p + jnp.sum(ep, axis=1, keepdims=True)

        s_own, s_p = lax.fori_loop(
            0, nq, keep_step, (s_own, jnp.zeros((T, 1), jnp.float32))
        )

        def yrecv_step(i, acc):
            y_rdma(i).wait_recv()
            e = ry_ref[i].astype(jnp.float32)
            return acc + jnp.sum(e, axis=1, keepdims=True)

        s = s_own + lax.fori_loop(0, nq, yrecv_step, s_p)
        s_ref[...] = jnp.broadcast_to(s, (T, 128))

        def drain(i, carry):
            x_rdma(i).wait_send()
            y_rdma(i).wait_send()
            return carry

        lax.fori_loop(0, nq, drain, 0)

    eloc, rx, ry, svec = pl.pallas_call(
        body_a,
        out_shape=(
            jax.ShapeDtypeStruct((n_chunks, T, CHUNK), jnp.bfloat16),
            jax.ShapeDtypeStruct((nq, T, CHUNK), jnp.bfloat16),
            jax.ShapeDtypeStruct((nq, T, CHUNK), jnp.bfloat16),
            jax.ShapeDtypeStruct((T, 128), jnp.float32),
        ),
        in_specs=[
            pl.BlockSpec(memory_space=pltpu.VMEM),
            pl.BlockSpec(memory_space=pl.ANY),
        ],
        out_specs=(
            pl.BlockSpec(memory_space=pltpu.VMEM),
            pl.BlockSpec(memory_space=pltpu.VMEM),
            pl.BlockSpec(memory_space=pltpu.VMEM),
            pl.BlockSpec(memory_space=pltpu.VMEM),
        ),
        scratch_shapes=[
            pltpu.VMEM((2, D, CHUNK), jnp.float32),
            pltpu.SemaphoreType.DMA((2,)),
            pltpu.SemaphoreType.DMA((nq,)),
            pltpu.SemaphoreType.DMA((nq,)),
            pltpu.SemaphoreType.DMA((nq,)),
            pltpu.SemaphoreType.DMA((nq,)),
        ],
        compiler_params=pltpu.CompilerParams(
            collective_id=0,
            vmem_limit_bytes=62 * 1024 * 1024,
        ),
    )(x, W)

    def body_b(eloc_ref, rx_ref, ry_ref, s_ref, out_ref):
        my_x = lax.axis_index("x")
        my_y = lax.axis_index("y")
        my_base = my_x * V
        partner_base = (1 - my_x) * V

        r = 1.0 / s_ref[:, 0:1]

        def norm_local(c, carry):
            sl = pl.ds(my_base + c * CHUNK, CHUNK)
            out_ref[:, sl] = eloc_ref[c].astype(jnp.float32) * r
            return carry

        lax.fori_loop(0, n_chunks, norm_local, 0)

        def norm_partner(i, carry):
            xc = pl.ds(partner_base + (my_y * nq + i) * CHUNK, CHUNK)
            out_ref[:, xc] = rx_ref[i].astype(jnp.float32) * r
            yc = pl.ds(partner_base + ((1 - my_y) * nq + i) * CHUNK, CHUNK)
            out_ref[:, yc] = ry_ref[i].astype(jnp.float32) * r
            return carry

        lax.fori_loop(0, nq, norm_partner, 0)

    return pl.pallas_call(
        body_b,
        out_shape=jax.ShapeDtypeStruct((T, 2 * V), jnp.float32),
        in_specs=[
            pl.BlockSpec(memory_space=pltpu.VMEM),
            pl.BlockSpec(memory_space=pltpu.VMEM),
            pl.BlockSpec(memory_space=pltpu.VMEM),
            pl.BlockSpec(memory_space=pltpu.VMEM),
        ],
        out_specs=pl.BlockSpec(memory_space=pltpu.VMEM),
        compiler_params=pltpu.CompilerParams(
            vmem_limit_bytes=62 * 1024 * 1024,
        ),
    )(eloc, rx, ry, svec)
